# baseline (speedup 1.0000x reference)
import jax
import jax.numpy as jnp
from jax import lax
from jax.experimental import pallas as pl
from jax.experimental.pallas import tpu as pltpu
from jax.experimental.pallas import tpu_sc as plsc

V = 1000000
B = 16384
F = 26
K = 16
NC, NS = 2, 16
NW = NC * NS
RPW = B // NW
EPW = RPW * F
BLK_ROWS = 64
NBLK = RPW // BLK_ROWS
BLK_E = BLK_ROWS * F
CH = 128
NCH = BLK_E // CH
GROUPS = RPW // K

TC_COLS = 4096
TC_GRID = -(-V // TC_COLS)
V8P = TC_GRID * (TC_COLS // 8)
VP = V8P * 8


def _relayout_body(vt_ref, out_ref):
    x = vt_ref[...]
    c8 = TC_COLS // 8
    r_ids = lax.broadcasted_iota(jnp.int32, (K, K), 0)
    c_ids = lax.broadcasted_iota(jnp.int32, (K, K), 1)
    eye = jnp.where(r_ids == c_ids, 1.0, 0.0)
    t0 = lax.dot_general(x, eye, (((0,), (0,)), ((), ())),
                         preferred_element_type=jnp.float32)
    for s in range(8):
        out_ref[:, s * K:(s + 1) * K] = t0[s * c8:(s + 1) * c8, :]


def _fm_body(idx_hbm, idxw_hbm, val_hbm, w_hbm, v128_hbm, b_hbm, out_hbm,
             idxv, idxo, valv, wbuf, vbuf0, vbuf1, dbuf, outv, bv,
             sem_v0, sem_v1, sem_w):
    wid = lax.axis_index("s") * NC + lax.axis_index("c")
    ebase = wid * EPW
    rbase = wid * RPW
    vtab = v128_hbm

    pltpu.sync_copy(idx_hbm.at[pl.ds(ebase, EPW)], idxv)
    pltpu.sync_copy(idxw_hbm.at[pl.ds(ebase, EPW)], idxo)
    pltpu.sync_copy(val_hbm.at[pl.ds(ebase, EPW)], valv)
    pltpu.sync_copy(b_hbm, bv)
    pltpu.sync_copy(v128_hbm.at[pl.ds(0, 2), pl.ds(0, 16)], vbuf0.at[pl.ds(0, 2)])

    vbufs = (vbuf0, vbuf1)
    sems = (sem_v0, sem_v1)
    iota = lax.iota(jnp.int32, K)

    def fire_v(blk):
        return []

    def fire_w(blk):
        hs = []
        for c in range(NCH):
            off = blk * BLK_E + c * CH
            hs.append(pltpu.async_copy(
                w_hbm.at[idxo.at[pl.ds(off, CH)]],
                wbuf.at[pl.ds(off, CH)], sem_w))
        return hs

    hv = fire_v(0)
    w_hs = fire_w(0)

    for blk in range(NBLK):
        hv_next = None
        if blk + 1 < NBLK:
            hv_next = fire_v(blk + 1)
            w_hs += fire_w(blk + 1)
        for h in hv:
            h.wait()
        buf = vbufs[blk % 2]

        def row_body(r, carry, blk=blk, buf=buf):
            e0 = blk * BLK_E + r * F
            va = valv[pl.ds(e0, K)]
            vb = valv[pl.ds(e0 + (F - K), K)]
            acc = jnp.zeros((K,), jnp.float32)
            acc2 = jnp.zeros((K,), jnp.float32)
            for f in range(F):
                x = buf[r * F + f, :]
                val = va[f] if f < K else vb[f - (F - K)]
                xe = x * val
                acc = acc + xe
                acc2 = acc2 + xe * xe
            d = acc * acc - acc2
            plsc.store_scatter(dbuf, [iota * RPW + (blk * BLK_ROWS + r)], d)
            return carry

        lax.fori_loop(0, BLK_ROWS, row_body, 0)
        hv = hv_next

    for h in w_hs:
        h.wait()
    bvec = bv[...]

    def grp_body(g, carry):
        pacc = jnp.zeros((K,), jnp.float32)
        for k in range(K):
            pacc = pacc + dbuf[pl.ds(k * RPW + g * K, K)]
        wacc = jnp.zeros((K,), jnp.float32)
        eidx0 = iota * F + g * (K * F)
        for f in range(F):
            eidx = eidx0 + f
            wacc = wacc + (plsc.load_gather(wbuf, [eidx]) *
                           plsc.load_gather(valv, [eidx]))
        logit = wacc + bvec + 0.5 * pacc
        y = 1.0 / (1.0 + jnp.exp(-logit))
        outv[pl.ds(g * K, K)] = y
        return carry

    lax.fori_loop(0, GROUPS, grp_body, 0)
    pltpu.sync_copy(outv, out_hbm.at[pl.ds(rbase, RPW)])


def kernel(values, indices, w, v, b):
    v16 = v.T

    idx_orig = indices.reshape(-1)
    idx_flat = ((idx_orig & ~4095) + ((idx_orig & 511) << 3)
                + ((idx_orig & 4095) >> 9))
    val_flat = values.reshape(-1)
    w_flat = w.reshape(-1)
    b16 = jnp.broadcast_to(b, (K,))
    mesh = plsc.VectorSubcoreMesh(core_axis_name="c", subcore_axis_name="s",
                                  num_cores=NC, num_subcores=NS)
    fm = pl.kernel(
        _fm_body,
        out_type=jax.ShapeDtypeStruct((B,), jnp.float32),
        mesh=mesh,
        compiler_params=pltpu.CompilerParams(needs_layout_passes=False,
                                             use_tc_tiling_on_sc=False),
        scratch_types=[
            pltpu.VMEM((EPW,), jnp.int32),
            pltpu.VMEM((EPW,), jnp.int32),
            pltpu.VMEM((EPW,), jnp.float32),
            pltpu.VMEM((EPW,), jnp.float32),
            pltpu.VMEM((BLK_E, K), jnp.float32),
            pltpu.VMEM((BLK_E, K), jnp.float32),
            pltpu.VMEM((K * RPW,), jnp.float32),
            pltpu.VMEM((RPW,), jnp.float32),
            pltpu.VMEM((K,), jnp.float32),
            pltpu.SemaphoreType.DMA,
            pltpu.SemaphoreType.DMA,
            pltpu.SemaphoreType.DMA,
        ],
    )
    return fm(idx_flat, idx_orig, val_flat, w_flat, v16, b16)


# speedup vs baseline: 4.6675x; 4.6675x over previous
import jax
import jax.numpy as jnp
from jax import lax
from jax.experimental import pallas as pl
from jax.experimental.pallas import tpu as pltpu
from jax.experimental.pallas import tpu_sc as plsc

V = 1000000
B = 16384
F = 26
K = 16
NC, NS = 2, 16
NW = NC * NS
RPW = B // NW
EPW = RPW * F
BLK_ROWS = 64
NBLK = RPW // BLK_ROWS
BLK_E = BLK_ROWS * F
CH = 128
NCH = BLK_E // CH
GROUPS = RPW // K

TC_COLS = 4096
TC_GRID = -(-V // TC_COLS)
V8P = TC_GRID * (TC_COLS // 8)
VP = V8P * 8


def _relayout_body(vt_ref, out_ref):
    x = vt_ref[...]
    c8 = TC_COLS // 8
    x8 = jnp.concatenate([x[:, s * c8:(s + 1) * c8] for s in range(8)],
                         axis=0)
    out_ref[...] = x8.T


def _fm_body(idx_hbm, idxw_hbm, val_hbm, w_hbm, v128_hbm, b_hbm, out_hbm,
             idxv, idxo, valv, wbuf, vbuf0, vbuf1, dbuf, outv, bv,
             sem_v0, sem_v1, sem_w):
    wid = lax.axis_index("s") * NC + lax.axis_index("c")
    ebase = wid * EPW
    rbase = wid * RPW
    vtab = v128_hbm

    pltpu.sync_copy(idx_hbm.at[pl.ds(ebase, EPW)], idxv)
    pltpu.sync_copy(idxw_hbm.at[pl.ds(ebase, EPW)], idxo)
    pltpu.sync_copy(val_hbm.at[pl.ds(ebase, EPW)], valv)
    pltpu.sync_copy(b_hbm, bv)

    vbufs = (vbuf0, vbuf1)
    sems = (sem_v0, sem_v1)
    iota = lax.iota(jnp.int32, K)

    def fire_v(blk):
        buf, sem = vbufs[blk % 2], sems[blk % 2]
        hs = []
        for c in range(NCH):
            off = blk * BLK_E + c * CH
            hs.append(pltpu.async_copy(
                vtab.at[idxv.at[pl.ds(off, CH)]],
                buf.at[pl.ds(c * CH, CH)], sem))
        return hs

    def fire_w(blk):
        hs = []
        for c in range(NCH):
            off = blk * BLK_E + c * CH
            hs.append(pltpu.async_copy(
                w_hbm.at[idxo.at[pl.ds(off, CH)]],
                wbuf.at[pl.ds(off, CH)], sem_w))
        return hs

    hv = fire_v(0)
    w_hs = fire_w(0)

    for blk in range(NBLK):
        hv_next = None
        if blk + 1 < NBLK:
            hv_next = fire_v(blk + 1)
            w_hs += fire_w(blk + 1)
        for h in hv:
            h.wait()
        buf = vbufs[blk % 2]

        def row_body(r, carry, blk=blk, buf=buf):
            e0 = blk * BLK_E + r * F
            va = valv[pl.ds(e0, K)]
            vb = valv[pl.ds(e0 + (F - K), K)]
            acc = jnp.zeros((K,), jnp.float32)
            acc2 = jnp.zeros((K,), jnp.float32)
            for f in range(F):
                x = buf[r * F + f, :]
                val = va[f] if f < K else vb[f - (F - K)]
                xe = x * val
                acc = acc + xe
                acc2 = acc2 + xe * xe
            d = acc * acc - acc2
            plsc.store_scatter(dbuf, [iota * RPW + (blk * BLK_ROWS + r)], d)
            return carry

        lax.fori_loop(0, BLK_ROWS, row_body, 0)
        hv = hv_next

    for h in w_hs:
        h.wait()
    bvec = bv[...]

    def grp_body(g, carry):
        pacc = jnp.zeros((K,), jnp.float32)
        for k in range(K):
            pacc = pacc + dbuf[pl.ds(k * RPW + g * K, K)]
        wacc = jnp.zeros((K,), jnp.float32)
        eidx0 = iota * F + g * (K * F)
        for f in range(F):
            eidx = eidx0 + f
            wacc = wacc + (plsc.load_gather(wbuf, [eidx]) *
                           plsc.load_gather(valv, [eidx]))
        logit = wacc + bvec + 0.5 * pacc
        y = 1.0 / (1.0 + jnp.exp(-logit))
        outv[pl.ds(g * K, K)] = y
        return carry

    lax.fori_loop(0, GROUPS, grp_body, 0)
    pltpu.sync_copy(outv, out_hbm.at[pl.ds(rbase, RPW)])


def kernel(values, indices, w, v, b):
    w_flat = w.T.reshape(-1)
    v128 = pl.pallas_call(
        _relayout_body,
        grid=(TC_GRID,),
        in_specs=[pl.BlockSpec((K, TC_COLS), lambda i: (0, i))],
        out_specs=pl.BlockSpec((TC_COLS // 8, 128), lambda i: (i, 0)),
        out_shape=jax.ShapeDtypeStruct((V8P, 128), jnp.float32),
        compiler_params=pltpu.CompilerParams(
            fuse_transposed_lhs_in_matmul=True),
    )(v.T)
    v16 = v128.reshape(VP, K)

    idx_orig = indices.reshape(-1)
    idx_flat = ((idx_orig & ~4095) + ((idx_orig & 511) << 3)
                + ((idx_orig & 4095) >> 9))
    val_flat = values.reshape(-1)
    b16 = jnp.broadcast_to(b, (K,))
    mesh = plsc.VectorSubcoreMesh(core_axis_name="c", subcore_axis_name="s",
                                  num_cores=NC, num_subcores=NS)
    fm = pl.kernel(
        _fm_body,
        out_type=jax.ShapeDtypeStruct((B,), jnp.float32),
        mesh=mesh,
        compiler_params=pltpu.CompilerParams(needs_layout_passes=False,
                                             use_tc_tiling_on_sc=False),
        scratch_types=[
            pltpu.VMEM((EPW,), jnp.int32),
            pltpu.VMEM((EPW,), jnp.int32),
            pltpu.VMEM((EPW,), jnp.float32),
            pltpu.VMEM((EPW,), jnp.float32),
            pltpu.VMEM((BLK_E, K), jnp.float32),
            pltpu.VMEM((BLK_E, K), jnp.float32),
            pltpu.VMEM((K * RPW,), jnp.float32),
            pltpu.VMEM((RPW,), jnp.float32),
            pltpu.VMEM((K,), jnp.float32),
            pltpu.SemaphoreType.DMA,
            pltpu.SemaphoreType.DMA,
            pltpu.SemaphoreType.DMA,
        ],
    )
    return fm(idx_flat, idx_orig, val_flat, w_flat, v16, b16)


# speedup vs baseline: 6.7513x; 1.4464x over previous
import jax
import jax.numpy as jnp
from jax import lax
from jax.experimental import pallas as pl
from jax.experimental.pallas import tpu as pltpu
from jax.experimental.pallas import tpu_sc as plsc

V = 1000000
B = 16384
F = 26
K = 16
NC, NS = 2, 16
NW = NC * NS
RPW = B // NW
EPW = RPW * F
BLK_ROWS = 64
NBLK = RPW // BLK_ROWS
BLK_E = BLK_ROWS * F
CH = 128
NCH = BLK_E // CH
GROUPS = RPW // K

TC_COLS = 16384
TC_GRID = -(-V // TC_COLS)
V8P = TC_GRID * (TC_COLS // 8)
VP = V8P * 8


def _relayout_body(vt_ref, out_ref):
    x = vt_ref[...]
    c8 = TC_COLS // 8
    x8 = jnp.concatenate([x[:, s * c8:(s + 1) * c8] for s in range(8)],
                         axis=0)
    out_ref[...] = x8.T


def _fm_body(idx_hbm, idxw_hbm, val_hbm, w_hbm, v128_hbm, b_hbm, out_hbm,
             idxv, idxo, valv, wbuf, vbuf0, vbuf1, dbuf, outv, bv,
             sem_v0, sem_v1, sem_w):
    wid = lax.axis_index("s") * NC + lax.axis_index("c")
    ebase = wid * EPW
    rbase = wid * RPW
    vtab = v128_hbm

    pltpu.sync_copy(idx_hbm.at[pl.ds(ebase, EPW)], idxv)
    pltpu.sync_copy(idxw_hbm.at[pl.ds(ebase, EPW)], idxo)
    pltpu.sync_copy(val_hbm.at[pl.ds(ebase, EPW)], valv)
    pltpu.sync_copy(b_hbm, bv)

    vbufs = (vbuf0, vbuf1)
    sems = (sem_v0, sem_v1)
    iota = lax.iota(jnp.int32, K)

    def fire_v(blk):
        buf, sem = vbufs[blk % 2], sems[blk % 2]
        hs = []
        for c in range(NCH):
            off = blk * BLK_E + c * CH
            hs.append(pltpu.async_copy(
                vtab.at[idxv.at[pl.ds(off, CH)]],
                buf.at[pl.ds(c * CH, CH)], sem))
        return hs

    def fire_w(blk):
        hs = []
        for c in range(NCH):
            off = blk * BLK_E + c * CH
            hs.append(pltpu.async_copy(
                w_hbm.at[idxo.at[pl.ds(off, CH)]],
                wbuf.at[pl.ds(off, CH)], sem_w))
        return hs

    hv = fire_v(0)
    w_hs = fire_w(0)

    for blk in range(NBLK):
        hv_next = None
        if blk + 1 < NBLK:
            hv_next = fire_v(blk + 1)
            w_hs += fire_w(blk + 1)
        for h in hv:
            h.wait()
        buf = vbufs[blk % 2]

        def row_body(r, carry, blk=blk, buf=buf):
            e0 = blk * BLK_E + r * F
            va = valv[pl.ds(e0, K)]
            vb = valv[pl.ds(e0 + (F - K), K)]
            acc = jnp.zeros((K,), jnp.float32)
            acc2 = jnp.zeros((K,), jnp.float32)
            for f in range(F):
                x = buf[r * F + f, :]
                val = va[f] if f < K else vb[f - (F - K)]
                xe = x * val
                acc = acc + xe
                acc2 = acc2 + xe * xe
            d = acc * acc - acc2
            plsc.store_scatter(dbuf, [iota * RPW + (blk * BLK_ROWS + r)], d)
            return carry

        lax.fori_loop(0, BLK_ROWS, row_body, 0)
        hv = hv_next

    for h in w_hs:
        h.wait()
    bvec = bv[...]

    def grp_body(g, carry):
        pacc = jnp.zeros((K,), jnp.float32)
        for k in range(K):
            pacc = pacc + dbuf[pl.ds(k * RPW + g * K, K)]
        wacc = jnp.zeros((K,), jnp.float32)
        eidx0 = iota * F + g * (K * F)
        for f in range(F):
            eidx = eidx0 + f
            wacc = wacc + (plsc.load_gather(wbuf, [eidx]) *
                           plsc.load_gather(valv, [eidx]))
        logit = wacc + bvec + 0.5 * pacc
        y = 1.0 / (1.0 + jnp.exp(-logit))
        outv[pl.ds(g * K, K)] = y
        return carry

    lax.fori_loop(0, GROUPS, grp_body, 0)
    pltpu.sync_copy(outv, out_hbm.at[pl.ds(rbase, RPW)])


def kernel(values, indices, w, v, b):
    w_flat = w.T.reshape(-1)
    v128 = pl.pallas_call(
        _relayout_body,
        grid=(TC_GRID,),
        in_specs=[pl.BlockSpec((K, TC_COLS), lambda i: (0, i))],
        out_specs=pl.BlockSpec((TC_COLS // 8, 128), lambda i: (i, 0)),
        out_shape=jax.ShapeDtypeStruct((V8P, 128), jnp.float32),
        compiler_params=pltpu.CompilerParams(
            fuse_transposed_lhs_in_matmul=True),
    )(v.T)
    v16 = v128.reshape(VP, K)

    c8 = TC_COLS // 8
    sh = c8.bit_length() - 1
    idx_orig = indices.reshape(-1)
    idx_flat = ((idx_orig & ~(TC_COLS - 1)) + ((idx_orig & (c8 - 1)) << 3)
                + ((idx_orig & (TC_COLS - 1)) >> sh))
    val_flat = values.reshape(-1)
    b16 = jnp.broadcast_to(b, (K,))
    mesh = plsc.VectorSubcoreMesh(core_axis_name="c", subcore_axis_name="s",
                                  num_cores=NC, num_subcores=NS)
    fm = pl.kernel(
        _fm_body,
        out_type=jax.ShapeDtypeStruct((B,), jnp.float32),
        mesh=mesh,
        compiler_params=pltpu.CompilerParams(needs_layout_passes=False,
                                             use_tc_tiling_on_sc=False),
        scratch_types=[
            pltpu.VMEM((EPW,), jnp.int32),
            pltpu.VMEM((EPW,), jnp.int32),
            pltpu.VMEM((EPW,), jnp.float32),
            pltpu.VMEM((EPW,), jnp.float32),
            pltpu.VMEM((BLK_E, K), jnp.float32),
            pltpu.VMEM((BLK_E, K), jnp.float32),
            pltpu.VMEM((K * RPW,), jnp.float32),
            pltpu.VMEM((RPW,), jnp.float32),
            pltpu.VMEM((K,), jnp.float32),
            pltpu.SemaphoreType.DMA,
            pltpu.SemaphoreType.DMA,
            pltpu.SemaphoreType.DMA,
        ],
    )
    return fm(idx_flat, idx_orig, val_flat, w_flat, v16, b16)
